# baseline (speedup 1.0000x reference)
import jax
from jax import lax
from jax.experimental import pallas as pl
from jax.experimental.pallas import tpu as pltpu
from jax.experimental.pallas import tpu_sc as plsc


def _sc_body(x_hbm, src_hbm, out_hbm):
    c = lax.axis_index("c")
    s = lax.axis_index("s")
    w = s * 2 + c

    @pl.when(w == 0)
    def _():
        pltpu.sync_copy(src_hbm, out_hbm.at[0])

    @pl.when(w != 0)
    def _():
        pltpu.sync_copy(x_hbm.at[w], out_hbm.at[w])


def kernel(x, src):
    mesh = plsc.VectorSubcoreMesh(core_axis_name="c", subcore_axis_name="s")
    return pl.kernel(
        _sc_body,
        out_type=jax.ShapeDtypeStruct(x.shape, x.dtype),
        mesh=mesh,
    )(x, src)


# speedup vs baseline: 39.8706x; 39.8706x over previous
import jax
import jax.numpy as jnp
from jax import lax
from jax.experimental import pallas as pl
from jax.experimental.pallas import tpu as pltpu
from jax.experimental.pallas import tpu_sc as plsc

ROWS = 16384
COLS = 128
CHUNK = 128
NBUF = 4
NCH = ROWS // CHUNK


def _copy_pipeline(src_ref, dst_ref, bufs, rsems, wsems):
    for b in range(NBUF):
        pltpu.make_async_copy(
            src_ref.at[pl.ds(b * CHUNK, CHUNK)], bufs[b], rsems[b]).start()

    def body(g, carry):
        for b in range(NBUF):
            i = g * NBUF + b
            sl = pl.ds(i * CHUNK, CHUNK)
            pltpu.make_async_copy(src_ref.at[sl], bufs[b], rsems[b]).wait()
            pltpu.make_async_copy(bufs[b], dst_ref.at[sl], wsems[b]).start()
            nxt = i + NBUF

            @pl.when(nxt < NCH)
            def _():
                pltpu.make_async_copy(bufs[b], dst_ref.at[sl], wsems[b]).wait()
                pltpu.make_async_copy(
                    src_ref.at[pl.ds(nxt * CHUNK, CHUNK)], bufs[b],
                    rsems[b]).start()
        return carry

    lax.fori_loop(0, NCH // NBUF, body, 0)
    for b in range(NBUF):
        i = NCH - NBUF + b
        sl = pl.ds(i * CHUNK, CHUNK)
        pltpu.make_async_copy(bufs[b], dst_ref.at[sl], wsems[b]).wait()


def _sc_body(x_hbm, src_hbm, out_hbm, b0, b1, b2, b3,
             r0, r1, r2, r3, w0, w1, w2, w3):
    c = lax.axis_index("c")
    s = lax.axis_index("s")
    w = s * 2 + c
    bufs = (b0, b1, b2, b3)
    rsems = (r0, r1, r2, r3)
    wsems = (w0, w1, w2, w3)

    @pl.when(w == 0)
    def _():
        _copy_pipeline(src_hbm, out_hbm.at[0], bufs, rsems, wsems)

    @pl.when(w != 0)
    def _():
        _copy_pipeline(x_hbm.at[w], out_hbm.at[w], bufs, rsems, wsems)


def kernel(x, src):
    mesh = plsc.VectorSubcoreMesh(core_axis_name="c", subcore_axis_name="s")
    return pl.kernel(
        _sc_body,
        out_type=jax.ShapeDtypeStruct(x.shape, x.dtype),
        mesh=mesh,
        scratch_types=(
            [pltpu.VMEM((CHUNK, COLS), jnp.float32) for _ in range(NBUF)]
            + [pltpu.SemaphoreType.DMA for _ in range(2 * NBUF)]
        ),
    )(x, src)
